# baseline (speedup 1.0000x reference)
import jax
import jax.numpy as jnp
from jax import lax
from jax.experimental import pallas as pl
from jax.experimental.pallas import tpu as pltpu
from jax.experimental.pallas import tpu_sc as plsc

_NUMROWS = 1000000
_D = 32
_BATCH = 16384
_COLS = 26
_B = _BATCH * _COLS
_NC = 2
_NS = 16
_NW = _NC * _NS
_BPW = _B // _NW
_S = 128
_G = 4
_CH = _S * _G
_NCH = _BPW // _CH
_JPW = _BPW // _S


def _sc_body(x2, rw2, iw2, out_re, out_im, idx_all, rows_r, rows_i,
             sem_r, sem_i):
    c = lax.axis_index("c")
    s = lax.axis_index("s")
    wid = s * _NC + c
    rw = rw2
    iw = iw2
    o_re = out_re
    o_im = out_im
    wrow = wid * _JPW
    wbase = wid * _BPW
    pltpu.sync_copy(x2.at[pl.ds(wrow, _JPW)], idx_all)

    @pl.loop(0, _NCH)
    def _chunk(ci):
        copies = []
        for g in range(_G):
            step = ci * _G + g
            copies.append(pltpu.async_copy(
                rw.at[idx_all.at[step]], rows_r.at[pl.ds(g * _S, _S)], sem_r))
            copies.append(pltpu.async_copy(
                iw.at[idx_all.at[step]], rows_i.at[pl.ds(g * _S, _S)], sem_i))
        for cp in copies:
            cp.wait()
        base = wbase + ci * _CH
        pltpu.sync_copy(rows_r, o_re.at[pl.ds(base, _CH)])
        pltpu.sync_copy(rows_i, o_im.at[pl.ds(base, _CH)])


def _gather_planar(x2, rw2, iw2):
    f = pl.kernel(
        _sc_body,
        out_type=(
            jax.ShapeDtypeStruct((_B, _D), jnp.float32),
            jax.ShapeDtypeStruct((_B, _D), jnp.float32),
        ),
        mesh=plsc.VectorSubcoreMesh(core_axis_name="c", subcore_axis_name="s"),
        scratch_types=[
            pltpu.VMEM((_JPW, _S), jnp.int32),
            pltpu.VMEM((_CH, _D), jnp.float32),
            pltpu.VMEM((_CH, _D), jnp.float32),
            pltpu.SemaphoreType.DMA,
            pltpu.SemaphoreType.DMA,
        ],
        compiler_params=pltpu.CompilerParams(use_tc_tiling_on_sc=False),
    )
    return f(x2, rw2, iw2)


def kernel(x, real_w, imag_w):
    x2 = x.reshape(_B // _S, _S)
    rw2 = real_w
    iw2 = imag_w
    re, im = _gather_planar(x2, rw2, iw2)
    return lax.complex(re.reshape(_BATCH, _COLS, _D),
                       im.reshape(_BATCH, _COLS, _D))


# speedup vs baseline: 1.1618x; 1.1618x over previous
import jax
import jax.numpy as jnp
from jax import lax
from jax.experimental import pallas as pl
from jax.experimental.pallas import tpu as pltpu
from jax.experimental.pallas import tpu_sc as plsc

_NUMROWS = 1000000
_D = 32
_BATCH = 16384
_COLS = 26
_B = _BATCH * _COLS
_NC = 2
_NS = 16
_NW = _NC * _NS
_RPW = _BATCH // _NW
_BPW = _B // _NW
_S = 128
_JPW = _BPW // _S
_CR = 64
_CH = _CR * _COLS
_GPC = _CH // _S
_NCH = _RPW // _CR


def _sc_body(x2, rw, iw, o_re, o_im, idx_all, rows_r, rows_i,
             sem_r, sem_i, sem_w):
    c = lax.axis_index("c")
    s = lax.axis_index("s")
    wid = s * _NC + c
    wrow = wid * _JPW
    pltpu.sync_copy(x2.at[pl.ds(wrow, _JPW)], idx_all)

    @pl.loop(0, _NCH)
    def _chunk(ci):
        @pl.loop(0, _GPC)
        def _issue(j):
            step = ci * _GPC + j
            pltpu.make_async_copy(
                rw.at[idx_all.at[step]],
                rows_r.at[pl.ds(j * _S, _S)], sem_r).start()
            pltpu.make_async_copy(
                iw.at[idx_all.at[step]],
                rows_i.at[pl.ds(j * _S, _S)], sem_i).start()

        @pl.loop(0, _GPC)
        def _drain(j):
            step = ci * _GPC + j
            pltpu.make_async_copy(
                rw.at[idx_all.at[step]],
                rows_r.at[pl.ds(j * _S, _S)], sem_r).wait()
            pltpu.make_async_copy(
                iw.at[idx_all.at[step]],
                rows_i.at[pl.ds(j * _S, _S)], sem_i).wait()

        b0 = wid * _RPW + ci * _CR

        @pl.loop(0, _CR)
        def _wr_issue(r):
            pltpu.make_async_copy(
                rows_r.at[pl.ds(r * _COLS, _COLS)], o_re.at[b0 + r],
                sem_w).start()
            pltpu.make_async_copy(
                rows_i.at[pl.ds(r * _COLS, _COLS)], o_im.at[b0 + r],
                sem_w).start()

        @pl.loop(0, _CR)
        def _wr_drain(r):
            pltpu.make_async_copy(
                rows_r.at[pl.ds(r * _COLS, _COLS)], o_re.at[b0 + r],
                sem_w).wait()
            pltpu.make_async_copy(
                rows_i.at[pl.ds(r * _COLS, _COLS)], o_im.at[b0 + r],
                sem_w).wait()


def _gather_planar(x2, rw, iw):
    f = pl.kernel(
        _sc_body,
        out_type=(
            jax.ShapeDtypeStruct((_BATCH, _COLS, _D), jnp.float32),
            jax.ShapeDtypeStruct((_BATCH, _COLS, _D), jnp.float32),
        ),
        mesh=plsc.VectorSubcoreMesh(core_axis_name="c", subcore_axis_name="s"),
        scratch_types=[
            pltpu.VMEM((_JPW, _S), jnp.int32),
            pltpu.VMEM((_CH, _D), jnp.float32),
            pltpu.VMEM((_CH, _D), jnp.float32),
            pltpu.SemaphoreType.DMA,
            pltpu.SemaphoreType.DMA,
            pltpu.SemaphoreType.DMA,
        ],
        compiler_params=pltpu.CompilerParams(use_tc_tiling_on_sc=False),
    )
    return f(x2, rw, iw)


def kernel(x, real_w, imag_w):
    x2 = x.reshape(_B // _S, _S)
    re, im = _gather_planar(x2, real_w, imag_w)
    return lax.complex(re, im)


# speedup vs baseline: 2.7238x; 2.3445x over previous
import jax
import jax.numpy as jnp
from jax import lax
from jax.experimental import pallas as pl
from jax.experimental.pallas import tpu as pltpu
from jax.experimental.pallas import tpu_sc as plsc

_NUMROWS = 1000000
_D = 32
_BATCH = 16384
_COLS = 26
_B = _BATCH * _COLS
_NC = 2
_NS = 16
_NW = _NC * _NS
_RPW = _BATCH // _NW
_S = 128
_NT = _RPW // _S
_DT = _D // 8
_BT = _BATCH // _S


def _sc_body(xT2, rw, iw, o_re, o_im, idx_b, rows_r, rows_i, tr_r, tr_i,
             sem_r, sem_i, sem_w):
    c = lax.axis_index("c")
    s = lax.axis_index("s")
    wid = s * _NC + c
    iota = lax.iota(jnp.int32, 16)

    @pl.loop(0, _COLS)
    def _col(j):
        pltpu.sync_copy(xT2.at[pl.ds(j * _BT + wid * _NT, _NT)], idx_b)
        for k in range(_NT):
            pltpu.make_async_copy(
                rw.at[idx_b.at[k]], rows_r.at[pl.ds(k * _S, _S)], sem_r
            ).start()
            pltpu.make_async_copy(
                iw.at[idx_b.at[k]], rows_i.at[pl.ds(k * _S, _S)], sem_i
            ).start()
        for k in range(_NT):
            pltpu.make_async_copy(
                rw.at[idx_b.at[k]], rows_r.at[pl.ds(k * _S, _S)], sem_r
            ).wait()
            pltpu.make_async_copy(
                iw.at[idx_b.at[k]], rows_i.at[pl.ds(k * _S, _S)], sem_i
            ).wait()

        @pl.loop(0, _NT)
        def _bt(k):
            @pl.loop(0, _S // 16)
            def _grp(k2):
                rowv = iota + (k * _S + k2 * 16)
                for d in range(_D):
                    dt, din = d // 8, d % 8
                    dv = jnp.full((16,), d, jnp.int32)
                    vr = plsc.load_gather(rows_r, [rowv, dv])
                    tr_r[dt, k, din, pl.ds(k2 * 16, 16)] = vr
                    vi = plsc.load_gather(rows_i, [rowv, dv])
                    tr_i[dt, k, din, pl.ds(k2 * 16, 16)] = vi

        for dt in range(_DT):
            pltpu.make_async_copy(
                tr_r.at[dt], o_re.at[j, dt, pl.ds(wid * _NT, _NT)], sem_w
            ).start()
            pltpu.make_async_copy(
                tr_i.at[dt], o_im.at[j, dt, pl.ds(wid * _NT, _NT)], sem_w
            ).start()
        for dt in range(_DT):
            pltpu.make_async_copy(
                tr_r.at[dt], o_re.at[j, dt, pl.ds(wid * _NT, _NT)], sem_w
            ).wait()
            pltpu.make_async_copy(
                tr_i.at[dt], o_im.at[j, dt, pl.ds(wid * _NT, _NT)], sem_w
            ).wait()


def _gather_planes(xT2, rw, iw):
    f = pl.kernel(
        _sc_body,
        out_type=(
            jax.ShapeDtypeStruct((_COLS, _DT, _BT, 8, _S), jnp.float32),
            jax.ShapeDtypeStruct((_COLS, _DT, _BT, 8, _S), jnp.float32),
        ),
        mesh=plsc.VectorSubcoreMesh(core_axis_name="c", subcore_axis_name="s"),
        scratch_types=[
            pltpu.VMEM((_NT, _S), jnp.int32),
            pltpu.VMEM((_RPW, _D), jnp.float32),
            pltpu.VMEM((_RPW, _D), jnp.float32),
            pltpu.VMEM((_DT, _NT, 8, _S), jnp.float32),
            pltpu.VMEM((_DT, _NT, 8, _S), jnp.float32),
            pltpu.SemaphoreType.DMA,
            pltpu.SemaphoreType.DMA,
            pltpu.SemaphoreType.DMA,
        ],
        compiler_params=pltpu.CompilerParams(use_tc_tiling_on_sc=False, needs_layout_passes=False),
    )
    return f(xT2, rw, iw)


def kernel(x, real_w, imag_w):
    xT2 = jnp.transpose(x).reshape(_B // _S, _S)
    re5, im5 = _gather_planes(xT2, real_w, imag_w)
    re3 = re5.transpose(2, 4, 0, 1, 3).reshape(_BATCH, _COLS, _D)
    im3 = im5.transpose(2, 4, 0, 1, 3).reshape(_BATCH, _COLS, _D)
    return lax.complex(re3, im3)


# speedup vs baseline: 3.3949x; 1.2464x over previous
import jax
import jax.numpy as jnp
from jax import lax
from jax.experimental import pallas as pl
from jax.experimental.pallas import tpu as pltpu
from jax.experimental.pallas import tpu_sc as plsc

_NUMROWS = 1000000
_D = 32
_BATCH = 16384
_COLS = 26
_B = _BATCH * _COLS
_NC = 2
_NS = 16
_NW = _NC * _NS
_RPW = _BATCH // _NW
_S = 128
_NT = _RPW // _S
_DT = _D // 8
_BT = _BATCH // _S
_PB = _S + 1


def _sc_body(xT2, rw, iw, o_re, o_im, idx_b, rows_r, rows_i, tr_r, tr_i,
             sem_r, sem_i, sem_w):
    c = lax.axis_index("c")
    s = lax.axis_index("s")
    wid = s * _NC + c
    iota = lax.iota(jnp.int32, 16)
    dinv = lax.rem(iota, 8)

    def stage_and_issue(j, buf):
        pltpu.sync_copy(xT2.at[pl.ds(j * _BT + wid * _NT, _NT)],
                        idx_b.at[buf])
        for k in range(_NT):
            pltpu.make_async_copy(
                rw.at[idx_b.at[buf, k]],
                rows_r.at[buf, pl.ds(k * _S, _S)], sem_r
            ).start()
            pltpu.make_async_copy(
                iw.at[idx_b.at[buf, k]],
                rows_i.at[buf, pl.ds(k * _S, _S)], sem_i
            ).start()

    def drain_gathers(j, buf):
        for k in range(_NT):
            pltpu.make_async_copy(
                rw.at[idx_b.at[buf, k]],
                rows_r.at[buf, pl.ds(k * _S, _S)], sem_r
            ).wait()
            pltpu.make_async_copy(
                iw.at[idx_b.at[buf, k]],
                rows_i.at[buf, pl.ds(k * _S, _S)], sem_i
            ).wait()

    def issue_writes(j):
        for dt in range(_DT):
            pltpu.make_async_copy(
                tr_r.at[dt, :, :, pl.ds(0, _S)],
                o_re.at[j, dt, pl.ds(wid * _NT, _NT)], sem_w
            ).start()
            pltpu.make_async_copy(
                tr_i.at[dt, :, :, pl.ds(0, _S)],
                o_im.at[j, dt, pl.ds(wid * _NT, _NT)], sem_w
            ).start()

    def drain_writes(j):
        for dt in range(_DT):
            pltpu.make_async_copy(
                tr_r.at[dt, :, :, pl.ds(0, _S)],
                o_re.at[j, dt, pl.ds(wid * _NT, _NT)], sem_w
            ).wait()
            pltpu.make_async_copy(
                tr_i.at[dt, :, :, pl.ds(0, _S)],
                o_im.at[j, dt, pl.ds(wid * _NT, _NT)], sem_w
            ).wait()

    stage_and_issue(0, 0)

    @pl.loop(0, _COLS)
    def _col(j):
        buf = lax.rem(j, 2)
        nbuf = lax.rem(j + 1, 2)

        @pl.when(j < _COLS - 1)
        def _prefetch():
            stage_and_issue(j + 1, nbuf)

        drain_gathers(j, buf)

        @pl.when(j > 0)
        def _drainw():
            drain_writes(j - 1)

        @pl.loop(0, _NT)
        def _bt(k):
            kks = jnp.full((16,), k, jnp.int32)

            @pl.loop(0, _S)
            def _bi(b_in):
                b = k * _S + b_in
                bb = jnp.full((16,), b_in, jnp.int32)
                for m in range(2):
                    dtv = lax.div(iota, 8) + 2 * m
                    vr = rows_r[buf, b, pl.ds(m * 16, 16)]
                    plsc.store_scatter(tr_r, [dtv, kks, dinv, bb], vr)
                    vi = rows_i[buf, b, pl.ds(m * 16, 16)]
                    plsc.store_scatter(tr_i, [dtv, kks, dinv, bb], vi)

        issue_writes(j)

    drain_writes(_COLS - 1)


def _gather_planes(xT2, rw, iw):
    f = pl.kernel(
        _sc_body,
        out_type=(
            jax.ShapeDtypeStruct((_COLS, _DT, _BT, 8, _S), jnp.float32),
            jax.ShapeDtypeStruct((_COLS, _DT, _BT, 8, _S), jnp.float32),
        ),
        mesh=plsc.VectorSubcoreMesh(core_axis_name="c", subcore_axis_name="s"),
        scratch_types=[
            pltpu.VMEM((2, _NT, _S), jnp.int32),
            pltpu.VMEM((2, _RPW, _D), jnp.float32),
            pltpu.VMEM((2, _RPW, _D), jnp.float32),
            pltpu.VMEM((_DT, _NT, 8, _PB), jnp.float32),
            pltpu.VMEM((_DT, _NT, 8, _PB), jnp.float32),
            pltpu.SemaphoreType.DMA,
            pltpu.SemaphoreType.DMA,
            pltpu.SemaphoreType.DMA,
        ],
        compiler_params=pltpu.CompilerParams(
            use_tc_tiling_on_sc=False, needs_layout_passes=False),
    )
    return f(xT2, rw, iw)


def kernel(x, real_w, imag_w):
    xT2 = jnp.transpose(x).reshape(_B // _S, _S)
    re5, im5 = _gather_planes(xT2, real_w, imag_w)
    re3 = re5.transpose(2, 4, 0, 1, 3).reshape(_BATCH, _COLS, _D)
    im3 = im5.transpose(2, 4, 0, 1, 3).reshape(_BATCH, _COLS, _D)
    return lax.complex(re3, im3)
